# Initial kernel scaffold; baseline (speedup 1.0000x reference)
import jax
import jax.numpy as jnp
from jax.experimental import pallas as pl


def kernel(x, edge_index, W1, b1, W2, b2, W_fc, b_fc):
    raise NotImplementedError("write your pallas kernel here")


# speedup vs baseline: 15.0942x; 15.0942x over previous
import functools

import jax
import jax.numpy as jnp
from jax import lax
from jax.experimental import pallas as pl
from jax.experimental.pallas import tpu as pltpu
from jax.experimental.pallas import tpu_sc as plsc

N_NODES = 10000
N_EDGES = 320000
D_IN = 128
D_H = 64

NC = 2
NS = 16
NW = NC * NS
E_PER_W = N_EDGES // NW
CHUNK = 80
N_CHUNKS = E_PER_W // CHUNK
N_PAD = 10240
ROWS_PER_TILE = N_PAD // NS

_mesh = plsc.VectorSubcoreMesh(core_axis_name="c", subcore_axis_name="s")


def _deg_body(dst_hbm, out_hbm, dst_v, ones_v, zb, deg_sh, sem):
    cid = lax.axis_index("c")
    sid = lax.axis_index("s")

    def fill16(i, ref, val):
        ref[pl.ds(i * 16, 16)] = jnp.full((16,), val, jnp.float32)

    for i in range(CHUNK // 16):
        fill16(i, ones_v, 1.0)
    for i in range(ROWS_PER_TILE // 16):
        fill16(i, zb, 0.0)
    pltpu.sync_copy(zb, deg_sh.at[pl.ds(sid * ROWS_PER_TILE, ROWS_PER_TILE)])
    plsc.subcore_barrier()

    ebase = (cid * NS + sid) * E_PER_W

    def body(i, carry):
        off = pl.multiple_of(ebase + i * CHUNK, 8)
        pltpu.sync_copy(dst_hbm.at[pl.ds(off, CHUNK)], dst_v)
        pltpu.sync_copy(ones_v, deg_sh.at[dst_v], add=True)
        return carry

    lax.fori_loop(0, N_CHUNKS, body, 0)
    plsc.subcore_barrier()
    roff = sid * ROWS_PER_TILE
    pltpu.sync_copy(deg_sh.at[pl.ds(roff, ROWS_PER_TILE)],
                    out_hbm.at[cid, pl.ds(roff, ROWS_PER_TILE)])


@functools.partial(
    pl.kernel,
    out_type=jax.ShapeDtypeStruct((NC, N_PAD), jnp.float32),
    mesh=_mesh,
    scratch_types=[
        pltpu.VMEM((CHUNK,), jnp.int32),
        pltpu.VMEM((CHUNK,), jnp.float32),
        pltpu.VMEM((ROWS_PER_TILE,), jnp.float32),
        pltpu.VMEM_SHARED((N_PAD,), jnp.float32),
        pltpu.SemaphoreType.DMA,
    ],
)
def _deg_kernel(dst_hbm, out_hbm, dst_v, ones_v, zb, deg_sh, sem):
    _deg_body(dst_hbm, out_hbm, dst_v, ones_v, zb, deg_sh, sem)


def _agg_body(g_hbm, src_hbm, dst_hbm, out_hbm, src_v, dst_v, rows_v, zb,
              y_sh, sem):
    cid = lax.axis_index("c")
    sid = lax.axis_index("s")

    for r in range(64):
        for j in range(D_H // 16):
            zb[r, pl.ds(j * 16, 16)] = jnp.zeros((16,), jnp.float32)
    for k in range(ROWS_PER_TILE // 64):
        pltpu.sync_copy(zb, y_sh.at[pl.ds(sid * ROWS_PER_TILE + k * 64, 64)])
    plsc.subcore_barrier()

    ebase = (cid * NS + sid) * E_PER_W

    def body(i, carry):
        off = pl.multiple_of(ebase + i * CHUNK, 8)
        pltpu.sync_copy(src_hbm.at[pl.ds(off, CHUNK)], src_v)
        pltpu.sync_copy(dst_hbm.at[pl.ds(off, CHUNK)], dst_v)
        pltpu.async_copy(g_hbm.at[src_v], rows_v, sem).wait()
        pltpu.sync_copy(rows_v, y_sh.at[dst_v], add=True)
        return carry

    lax.fori_loop(0, N_CHUNKS, body, 0)
    plsc.subcore_barrier()
    roff = sid * ROWS_PER_TILE
    pltpu.sync_copy(y_sh.at[pl.ds(roff, ROWS_PER_TILE)],
                    out_hbm.at[cid, pl.ds(roff, ROWS_PER_TILE)])


@functools.partial(
    pl.kernel,
    out_type=jax.ShapeDtypeStruct((NC, N_PAD, D_H), jnp.float32),
    mesh=_mesh,
    scratch_types=[
        pltpu.VMEM((CHUNK,), jnp.int32),
        pltpu.VMEM((CHUNK,), jnp.int32),
        pltpu.VMEM((CHUNK, D_H), jnp.float32),
        pltpu.VMEM((64, D_H), jnp.float32),
        pltpu.VMEM_SHARED((N_PAD, D_H), jnp.float32),
        pltpu.SemaphoreType.DMA,
    ],
    compiler_params=pltpu.CompilerParams(use_tc_tiling_on_sc=False),
)
def _agg_kernel(g_hbm, src_hbm, dst_hbm, out_hbm, src_v, dst_v, rows_v, zb,
                y_sh, sem):
    _agg_body(g_hbm, src_hbm, dst_hbm, out_hbm, src_v, dst_v, rows_v, zb,
              y_sh, sem)


def _tc1_body(x_ref, w1_ref, degp_ref, g1_ref, dinv_ref):
    deg = degp_ref[0] + degp_ref[1] + 1.0
    dinv = lax.rsqrt(deg)
    dinv_ref[...] = dinv
    h = jnp.dot(x_ref[...], w1_ref[...], preferred_element_type=jnp.float32)
    g1_ref[...] = h * dinv[:N_NODES]


def _tc1(x, w1, degp_col):
    return pl.pallas_call(
        _tc1_body,
        out_shape=(
            jax.ShapeDtypeStruct((N_NODES, D_H), jnp.float32),
            jax.ShapeDtypeStruct((N_PAD, 1), jnp.float32),
        ),
    )(x, w1, degp_col)


def _tc2_body(yp_ref, g1_ref, dinv_ref, b1_ref, w2_ref, g2_ref):
    ysum = yp_ref[0, :N_NODES] + yp_ref[1, :N_NODES] + g1_ref[...]
    dinv = dinv_ref[:N_NODES]
    h1 = jax.nn.relu(ysum * dinv + b1_ref[...])
    h2 = jnp.dot(h1, w2_ref[...], preferred_element_type=jnp.float32)
    g2_ref[...] = h2 * dinv


def _tc2(yp, g1, dinv, b1_row, w2):
    return pl.pallas_call(
        _tc2_body,
        out_shape=jax.ShapeDtypeStruct((N_NODES, D_H), jnp.float32),
    )(yp, g1, dinv, b1_row, w2)


def _tc3_body(yp_ref, g2_ref, dinv_ref, b2_ref, wfc_ref, bfc_ref, out_ref):
    ysum = yp_ref[0, :N_NODES] + yp_ref[1, :N_NODES] + g2_ref[...]
    dinv = dinv_ref[:N_NODES]
    h2 = jax.nn.relu(ysum * dinv + b2_ref[...])
    m = jnp.sum(h2, axis=1, keepdims=True) * (1.0 / D_H)
    out_ref[...] = jnp.sum(m * wfc_ref[...], keepdims=True) + bfc_ref[...]


def _tc3(yp, g2, dinv, b2_row, w_fc, b_fc_2d):
    return pl.pallas_call(
        _tc3_body,
        out_shape=jax.ShapeDtypeStruct((1, 1), jnp.float32),
    )(yp, g2, dinv, b2_row, w_fc, b_fc_2d)


@jax.jit
def kernel(x, edge_index, W1, b1, W2, b2, W_fc, b_fc):
    src = edge_index[0]
    dst = edge_index[1]

    degp = _deg_kernel(dst)
    degp_col = degp.reshape(NC, N_PAD, 1)
    g1, dinv = _tc1(x, W1, degp_col)

    yp1 = _agg_kernel(g1, src, dst)
    g2 = _tc2(yp1, g1, dinv, b1.reshape(1, D_H), W2)

    yp2 = _agg_kernel(g2, src, dst)
    out = _tc3(yp2, g2, dinv, b2.reshape(1, D_H), W_fc, b_fc.reshape(1, 1))
    return out


# speedup vs baseline: 43.1948x; 2.8617x over previous
import functools

import jax
import jax.numpy as jnp
from jax import lax
from jax.experimental import pallas as pl
from jax.experimental.pallas import tpu as pltpu
from jax.experimental.pallas import tpu_sc as plsc

N_NODES = 10000
N_EDGES = 320000
D_IN = 128
D_H = 64

NC = 2
NS = 16
NW = NC * NS
E_PER_W = N_EDGES // NW
CHUNK = 80
CPW = E_PER_W // CHUNK
N_PAD = 10240
ROWS_PER_TILE = N_PAD // NS

_mesh = plsc.VectorSubcoreMesh(core_axis_name="c", subcore_axis_name="s")
_sc_params = pltpu.CompilerParams(use_tc_tiling_on_sc=False)


DEG_G = 25
DEG_NG = CPW // DEG_G


def _deg_body(dst_hbm, out_hbm, dstb, ones_v, zb, deg_sh, sem_i, sem_s):
    cid = lax.axis_index("c")
    sid = lax.axis_index("s")

    for i in range(CHUNK // 16):
        ones_v[pl.ds(i * 16, 16)] = jnp.full((16,), 1.0, jnp.float32)
    for i in range(ROWS_PER_TILE // 16):
        zb[pl.ds(i * 16, 16)] = jnp.zeros((16,), jnp.float32)
    pltpu.sync_copy(zb, deg_sh.at[pl.ds(sid * ROWS_PER_TILE, ROWS_PER_TILE)])
    plsc.subcore_barrier()

    crow = (cid * NS + sid) * CPW

    def idx_copy(g, slot):
        return pltpu.make_async_copy(
            dst_hbm.at[pl.ds(crow + g * DEG_G, DEG_G)], dstb.at[slot], sem_i)

    idx_copy(0, 0).start()
    idx_copy(0, 0).wait()
    idx_copy(1, 1).start()

    def body(g, carry):
        p = g % 2
        q = 1 - p

        @pl.when(g + 1 < DEG_NG)
        def _():
            idx_copy(g + 1, q).wait()

        for b in range(DEG_G):
            pltpu.async_copy(ones_v, deg_sh.at[dstb.at[p, b]], sem_s, add=True)
        for b in range(DEG_G):
            pltpu.make_async_copy(ones_v, deg_sh.at[dstb.at[p, b]],
                                  sem_s).wait()

        @pl.when(g + 2 < DEG_NG)
        def _():
            idx_copy(g + 2, p).start()

        return carry

    lax.fori_loop(0, DEG_NG, body, 0)
    plsc.subcore_barrier()
    roff = sid * ROWS_PER_TILE
    pltpu.sync_copy(deg_sh.at[pl.ds(roff, ROWS_PER_TILE)],
                    out_hbm.at[cid, pl.ds(roff, ROWS_PER_TILE)])


@functools.partial(
    pl.kernel,
    out_type=jax.ShapeDtypeStruct((NC, N_PAD), jnp.float32),
    mesh=_mesh,
    scratch_types=[
        pltpu.VMEM((2, DEG_G, CHUNK), jnp.int32),
        pltpu.VMEM((CHUNK,), jnp.float32),
        pltpu.VMEM((ROWS_PER_TILE,), jnp.float32),
        pltpu.VMEM_SHARED((N_PAD,), jnp.float32),
        pltpu.SemaphoreType.DMA,
        pltpu.SemaphoreType.DMA,
    ],
    compiler_params=_sc_params,
)
def _deg_kernel(dst_hbm, out_hbm, dstb, ones_v, zb, deg_sh, sem_i, sem_s):
    _deg_body(dst_hbm, out_hbm, dstb, ones_v, zb, deg_sh, sem_i, sem_s)


AGG_G = 5
AGG_NG = CPW // AGG_G


def _agg_body(g_hbm, src_hbm, dst_hbm, out_hbm, srcb, dstb, rows, zb, y_sh,
              sem_i, sem_g, sem_s):
    cid = lax.axis_index("c")
    sid = lax.axis_index("s")

    for r in range(64):
        for j in range(D_H // 16):
            zb[r, pl.ds(j * 16, 16)] = jnp.zeros((16,), jnp.float32)
    for k in range(ROWS_PER_TILE // 64):
        pltpu.sync_copy(zb, y_sh.at[pl.ds(sid * ROWS_PER_TILE + k * 64, 64)])
    plsc.subcore_barrier()

    crow = (cid * NS + sid) * CPW

    def src_copy(g, slot):
        return pltpu.make_async_copy(
            src_hbm.at[pl.ds(crow + g * AGG_G, AGG_G)], srcb.at[slot], sem_i)

    def dst_copy(g, slot):
        return pltpu.make_async_copy(
            dst_hbm.at[pl.ds(crow + g * AGG_G, AGG_G)], dstb.at[slot], sem_i)

    def gather(slot, b):
        return pltpu.make_async_copy(
            g_hbm.at[srcb.at[slot, b]],
            rows.at[slot, pl.ds(b * CHUNK, CHUNK)], sem_g)

    def scatter(slot, b):
        return pltpu.async_copy(
            rows.at[slot, pl.ds(b * CHUNK, CHUNK)],
            y_sh.at[dstb.at[slot, b]], sem_s, add=True)

    def scatter_drain(slot, b):
        return pltpu.make_async_copy(
            rows.at[slot, pl.ds(b * CHUNK, CHUNK)],
            y_sh.at[dstb.at[slot, b]], sem_s)

    src_copy(0, 0).start(); dst_copy(0, 0).start()
    src_copy(0, 0).wait(); dst_copy(0, 0).wait()
    for b in range(AGG_G):
        gather(0, b).start()
    src_copy(1, 1).start(); dst_copy(1, 1).start()

    def body(g, carry):
        p = g % 2
        q = 1 - p

        @pl.when(g + 1 < AGG_NG)
        def _():
            src_copy(g + 1, q).wait()
            dst_copy(g + 1, q).wait()
            for b in range(AGG_G):
                gather(q, b).start()

        for b in range(AGG_G):
            gather(p, b).wait()
        for b in range(AGG_G):
            scatter(p, b)
        for b in range(AGG_G):
            scatter_drain(p, b).wait()

        @pl.when(g + 2 < AGG_NG)
        def _():
            src_copy(g + 2, p).start()
            dst_copy(g + 2, p).start()

        return carry

    lax.fori_loop(0, AGG_NG, body, 0)
    plsc.subcore_barrier()
    roff = sid * ROWS_PER_TILE
    pltpu.sync_copy(y_sh.at[pl.ds(roff, ROWS_PER_TILE)],
                    out_hbm.at[cid, pl.ds(roff, ROWS_PER_TILE)])


@functools.partial(
    pl.kernel,
    out_type=jax.ShapeDtypeStruct((NC, N_PAD, D_H), jnp.float32),
    mesh=_mesh,
    scratch_types=[
        pltpu.VMEM((2, AGG_G, CHUNK), jnp.int32),
        pltpu.VMEM((2, AGG_G, CHUNK), jnp.int32),
        pltpu.VMEM((2, AGG_G * CHUNK, D_H), jnp.float32),
        pltpu.VMEM((64, D_H), jnp.float32),
        pltpu.VMEM_SHARED((N_PAD, D_H), jnp.float32),
        pltpu.SemaphoreType.DMA,
        pltpu.SemaphoreType.DMA,
        pltpu.SemaphoreType.DMA,
    ],
    compiler_params=_sc_params,
)
def _agg_kernel(g_hbm, src_hbm, dst_hbm, out_hbm, srcb, dstb, rows, zb, y_sh,
                sem_i, sem_g, sem_s):
    _agg_body(g_hbm, src_hbm, dst_hbm, out_hbm, srcb, dstb, rows, zb, y_sh,
              sem_i, sem_g, sem_s)


def _tc0_body(x_ref, w1_ref, h_ref):
    h_ref[...] = jnp.dot(x_ref[...], w1_ref[...],
                         preferred_element_type=jnp.float32)


def _tc0(x, w1):
    return pl.pallas_call(
        _tc0_body,
        out_shape=jax.ShapeDtypeStruct((N_NODES, D_H), jnp.float32),
    )(x, w1)


def _tc1_body(h_ref, degp_ref, g1_ref, dinv_ref):
    deg = degp_ref[0] + degp_ref[1] + 1.0
    dinv = lax.rsqrt(deg)
    dinv_ref[...] = dinv
    g1_ref[...] = h_ref[...] * dinv[:N_NODES]


def _tc1(h1, degp_col):
    return pl.pallas_call(
        _tc1_body,
        out_shape=(
            jax.ShapeDtypeStruct((N_NODES, D_H), jnp.float32),
            jax.ShapeDtypeStruct((N_PAD, 1), jnp.float32),
        ),
    )(h1, degp_col)


def _tc2_body(yp_ref, g1_ref, dinv_ref, b1_ref, w2_ref, g2_ref):
    ysum = yp_ref[0, :N_NODES] + yp_ref[1, :N_NODES] + g1_ref[...]
    dinv = dinv_ref[:N_NODES]
    h1 = jax.nn.relu(ysum * dinv + b1_ref[...])
    h2 = jnp.dot(h1, w2_ref[...], preferred_element_type=jnp.float32)
    g2_ref[...] = h2 * dinv


def _tc2(yp, g1, dinv, b1_row, w2):
    return pl.pallas_call(
        _tc2_body,
        out_shape=jax.ShapeDtypeStruct((N_NODES, D_H), jnp.float32),
    )(yp, g1, dinv, b1_row, w2)


def _tc3_body(yp_ref, g2_ref, dinv_ref, b2_ref, wfc_ref, bfc_ref, out_ref):
    ysum = yp_ref[0, :N_NODES] + yp_ref[1, :N_NODES] + g2_ref[...]
    dinv = dinv_ref[:N_NODES]
    h2 = jax.nn.relu(ysum * dinv + b2_ref[...])
    m = jnp.sum(h2, axis=1, keepdims=True) * (1.0 / D_H)
    out_ref[...] = jnp.sum(m * wfc_ref[...], keepdims=True) + bfc_ref[...]


def _tc3(yp, g2, dinv, b2_row, w_fc, b_fc_2d):
    return pl.pallas_call(
        _tc3_body,
        out_shape=jax.ShapeDtypeStruct((1, 1), jnp.float32),
    )(yp, g2, dinv, b2_row, w_fc, b_fc_2d)


@jax.jit
def kernel(x, edge_index, W1, b1, W2, b2, W_fc, b_fc):
    src = edge_index[0].reshape(N_EDGES // CHUNK, CHUNK)
    dst = edge_index[1].reshape(N_EDGES // CHUNK, CHUNK)

    h1lin = _tc0(x, W1)
    degp = _deg_kernel(dst)
    g1, dinv = _tc1(h1lin, degp.reshape(NC, N_PAD, 1))

    yp1 = _agg_kernel(g1, src, dst)
    g2 = _tc2(yp1, g1, dinv, b1.reshape(1, D_H), W2)

    yp2 = _agg_kernel(g2, src, dst)
    out = _tc3(yp2, g2, dinv, b2.reshape(1, D_H), W_fc, b_fc.reshape(1, 1))
    return out


# speedup vs baseline: 43.3888x; 1.0045x over previous
import functools

import jax
import jax.numpy as jnp
from jax import lax
from jax.experimental import pallas as pl
from jax.experimental.pallas import tpu as pltpu
from jax.experimental.pallas import tpu_sc as plsc

N_NODES = 10000
N_EDGES = 320000
D_IN = 128
D_H = 64

NC = 2
NS = 16
NW = NC * NS
E_PER_W = N_EDGES // NW
CHUNK = 80
CPW = E_PER_W // CHUNK
N_PAD = 10240
ROWS_PER_TILE = N_PAD // NS

_mesh = plsc.VectorSubcoreMesh(core_axis_name="c", subcore_axis_name="s")
_sc_params = pltpu.CompilerParams(use_tc_tiling_on_sc=False)


DEG_G = 25
DEG_NG = CPW // DEG_G


def _deg_body(dst_hbm, out_hbm, dstb, ones_v, zb, deg_sh, sem_i, sem_s):
    cid = lax.axis_index("c")
    sid = lax.axis_index("s")

    for i in range(CHUNK // 16):
        ones_v[pl.ds(i * 16, 16)] = jnp.full((16,), 1.0, jnp.float32)
    for i in range(ROWS_PER_TILE // 16):
        zb[pl.ds(i * 16, 16)] = jnp.zeros((16,), jnp.float32)
    pltpu.sync_copy(zb, deg_sh.at[pl.ds(sid * ROWS_PER_TILE, ROWS_PER_TILE)])
    plsc.subcore_barrier()

    crow = (cid * NS + sid) * CPW

    def idx_copy(g, slot):
        return pltpu.make_async_copy(
            dst_hbm.at[pl.ds(crow + g * DEG_G, DEG_G)], dstb.at[slot], sem_i)

    idx_copy(0, 0).start()
    idx_copy(0, 0).wait()
    idx_copy(1, 1).start()

    def body(g, carry):
        p = g % 2
        q = 1 - p

        @pl.when(g + 1 < DEG_NG)
        def _():
            idx_copy(g + 1, q).wait()

        for b in range(DEG_G):
            pltpu.async_copy(ones_v, deg_sh.at[dstb.at[p, b]], sem_s, add=True)
        for b in range(DEG_G):
            pltpu.make_async_copy(ones_v, deg_sh.at[dstb.at[p, b]],
                                  sem_s).wait()

        @pl.when(g + 2 < DEG_NG)
        def _():
            idx_copy(g + 2, p).start()

        return carry

    lax.fori_loop(0, DEG_NG, body, 0)
    plsc.subcore_barrier()
    roff = sid * ROWS_PER_TILE
    pltpu.sync_copy(deg_sh.at[pl.ds(roff, ROWS_PER_TILE)],
                    out_hbm.at[cid, pl.ds(roff, ROWS_PER_TILE)])


@functools.partial(
    pl.kernel,
    out_type=jax.ShapeDtypeStruct((NC, N_PAD), jnp.float32),
    mesh=_mesh,
    scratch_types=[
        pltpu.VMEM((2, DEG_G, CHUNK), jnp.int32),
        pltpu.VMEM((CHUNK,), jnp.float32),
        pltpu.VMEM((ROWS_PER_TILE,), jnp.float32),
        pltpu.VMEM_SHARED((N_PAD,), jnp.float32),
        pltpu.SemaphoreType.DMA,
        pltpu.SemaphoreType.DMA,
    ],
    compiler_params=_sc_params,
)
def _deg_kernel(dst_hbm, out_hbm, dstb, ones_v, zb, deg_sh, sem_i, sem_s):
    _deg_body(dst_hbm, out_hbm, dstb, ones_v, zb, deg_sh, sem_i, sem_s)


AGG_G = 5
AGG_NG = CPW // AGG_G


def _agg_body(g_hbm, src_hbm, dst_hbm, out_hbm, srcb, dstb, rows, zb, y_sh,
              sem_i, sem_g, sem_s):
    cid = lax.axis_index("c")
    sid = lax.axis_index("s")

    for r in range(64):
        for j in range(D_H // 16):
            zb[r, pl.ds(j * 16, 16)] = jnp.zeros((16,), jnp.float32)
    for k in range(ROWS_PER_TILE // 64):
        pltpu.sync_copy(zb, y_sh.at[pl.ds(sid * ROWS_PER_TILE + k * 64, 64)])
    plsc.subcore_barrier()

    crow = (cid * NS + sid) * CPW

    def src_copy(g, slot):
        return pltpu.make_async_copy(
            src_hbm.at[pl.ds(crow + g * AGG_G, AGG_G)], srcb.at[slot], sem_i)

    def dst_copy(g, slot):
        return pltpu.make_async_copy(
            dst_hbm.at[pl.ds(crow + g * AGG_G, AGG_G)], dstb.at[slot], sem_i)

    def gather(slot, b):
        return pltpu.make_async_copy(
            g_hbm.at[srcb.at[slot, b]],
            rows.at[slot, pl.ds(b * CHUNK, CHUNK)], sem_g)

    def scatter(slot, b):
        return pltpu.async_copy(
            rows.at[slot, pl.ds(b * CHUNK, CHUNK)],
            y_sh.at[dstb.at[slot, b]], sem_s, add=True)

    def scatter_drain(slot, b):
        return pltpu.make_async_copy(
            rows.at[slot, pl.ds(b * CHUNK, CHUNK)],
            y_sh.at[dstb.at[slot, b]], sem_s)

    src_copy(0, 0).start(); dst_copy(0, 0).start()
    src_copy(0, 0).wait(); dst_copy(0, 0).wait()
    for b in range(AGG_G):
        gather(0, b).start()
    src_copy(1, 1).start(); dst_copy(1, 1).start()

    def body(g, carry):
        p = g % 2
        q = 1 - p

        @pl.when(g + 1 < AGG_NG)
        def _():
            src_copy(g + 1, q).wait()
            dst_copy(g + 1, q).wait()
            for b in range(AGG_G):
                gather(q, b).start()

        for b in range(AGG_G):
            gather(p, b).wait()
        for b in range(AGG_G):
            scatter(p, b)
        for b in range(AGG_G):
            scatter_drain(p, b).wait()

        @pl.when(g + 2 < AGG_NG)
        def _():
            src_copy(g + 2, p).start()
            dst_copy(g + 2, p).start()

        return carry

    lax.fori_loop(0, AGG_NG, body, 0)
    plsc.subcore_barrier()
    roff = sid * ROWS_PER_TILE
    pltpu.sync_copy(y_sh.at[pl.ds(roff, ROWS_PER_TILE)],
                    out_hbm.at[cid, pl.ds(roff, ROWS_PER_TILE)])


@functools.partial(
    pl.kernel,
    out_type=jax.ShapeDtypeStruct((NC, N_PAD, D_H), jnp.float32),
    mesh=_mesh,
    scratch_types=[
        pltpu.VMEM((2, AGG_G, CHUNK), jnp.int32),
        pltpu.VMEM((2, AGG_G, CHUNK), jnp.int32),
        pltpu.VMEM((2, AGG_G * CHUNK, D_H), jnp.float32),
        pltpu.VMEM((64, D_H), jnp.float32),
        pltpu.VMEM_SHARED((N_PAD, D_H), jnp.float32),
        pltpu.SemaphoreType.DMA,
        pltpu.SemaphoreType.DMA,
        pltpu.SemaphoreType.DMA,
    ],
    compiler_params=_sc_params,
)
def _agg_kernel(g_hbm, src_hbm, dst_hbm, out_hbm, srcb, dstb, rows, zb, y_sh,
                sem_i, sem_g, sem_s):
    _agg_body(g_hbm, src_hbm, dst_hbm, out_hbm, srcb, dstb, rows, zb, y_sh,
              sem_i, sem_g, sem_s)


def _tc1_body(x_ref, w1_ref, degp_ref, g1_ref, dinv_ref):
    deg = degp_ref[0] + degp_ref[1] + 1.0
    dinv = lax.rsqrt(deg)
    dinv_ref[...] = dinv
    xs = x_ref[...] * dinv[:N_NODES]
    g1_ref[...] = jnp.dot(xs, w1_ref[...], preferred_element_type=jnp.float32)


def _tc1(x, w1, degp_col):
    return pl.pallas_call(
        _tc1_body,
        out_shape=(
            jax.ShapeDtypeStruct((N_NODES, D_H), jnp.float32),
            jax.ShapeDtypeStruct((N_PAD, 1), jnp.float32),
        ),
    )(x, w1, degp_col)


def _tc2_body(yp_ref, g1_ref, dinv_ref, b1_ref, w2_ref, g2_ref):
    ysum = yp_ref[0, :N_NODES] + yp_ref[1, :N_NODES] + g1_ref[...]
    dinv = dinv_ref[:N_NODES]
    h1 = jax.nn.relu(ysum * dinv + b1_ref[...])
    h2 = jnp.dot(h1, w2_ref[...], preferred_element_type=jnp.float32)
    g2_ref[...] = h2 * dinv


def _tc2(yp, g1, dinv, b1_row, w2):
    return pl.pallas_call(
        _tc2_body,
        out_shape=jax.ShapeDtypeStruct((N_NODES, D_H), jnp.float32),
    )(yp, g1, dinv, b1_row, w2)


def _tc3_body(yp_ref, g2_ref, dinv_ref, b2_ref, wfc_ref, bfc_ref, out_ref):
    ysum = yp_ref[0, :N_NODES] + yp_ref[1, :N_NODES] + g2_ref[...]
    dinv = dinv_ref[:N_NODES]
    h2 = jax.nn.relu(ysum * dinv + b2_ref[...])
    m = jnp.sum(h2, axis=1, keepdims=True) * (1.0 / D_H)
    out_ref[...] = jnp.sum(m * wfc_ref[...], keepdims=True) + bfc_ref[...]


def _tc3(yp, g2, dinv, b2_row, w_fc, b_fc_2d):
    return pl.pallas_call(
        _tc3_body,
        out_shape=jax.ShapeDtypeStruct((1, 1), jnp.float32),
    )(yp, g2, dinv, b2_row, w_fc, b_fc_2d)


@jax.jit
def kernel(x, edge_index, W1, b1, W2, b2, W_fc, b_fc):
    src = edge_index[0].reshape(N_EDGES // CHUNK, CHUNK)
    dst = edge_index[1].reshape(N_EDGES // CHUNK, CHUNK)

    degp = _deg_kernel(dst)
    g1, dinv = _tc1(x, W1, degp.reshape(NC, N_PAD, 1))

    yp1 = _agg_kernel(g1, src, dst)
    g2 = _tc2(yp1, g1, dinv, b1.reshape(1, D_H), W2)

    yp2 = _agg_kernel(g2, src, dst)
    out = _tc3(yp2, g2, dinv, b2.reshape(1, D_H), W_fc, b_fc.reshape(1, 1))
    return out
